# baseline (speedup 1.0000x reference)
import functools

import jax
import jax.numpy as jnp
from jax import lax
from jax.experimental import pallas as pl
from jax.experimental.pallas import tpu as pltpu
from jax.experimental.pallas import tpu_sc as plsc

B = 8
S = 128
T = 64
U = 512
M = 2 * U
V = 8020


@functools.lru_cache(maxsize=None)
def _make_sc_gather():
    info = plsc.get_sparse_core_info()
    nw = info.num_cores * info.num_subcores
    sw = S * B // nw
    tw = T * B // nw
    mesh = plsc.VectorSubcoreMesh(core_axis_name="c", subcore_axis_name="s")

    @functools.partial(
        pl.kernel,
        mesh=mesh,
        out_type=(
            jax.ShapeDtypeStruct((S * B, U), jnp.float32),
            jax.ShapeDtypeStruct((T * B, U), jnp.float32),
        ),
        scratch_types=[
            pltpu.VMEM((sw,), jnp.int32),
            pltpu.VMEM((sw, U), jnp.float32),
            pltpu.VMEM((tw,), jnp.int32),
            pltpu.VMEM((tw, U), jnp.float32),
            pltpu.SemaphoreType.DMA,
            pltpu.SemaphoreType.DMA,
        ],
    )
    def gather(wtab_hbm, sidx_hbm, ctab_hbm, tidx_hbm, xsrc_hbm, xtgt_hbm,
               sidx_v, srows_v, tidx_v, trows_v, sem_s, sem_t):
        wid = lax.axis_index("s") * info.num_cores + lax.axis_index("c")
        sb = wid * sw
        tb = wid * tw
        pltpu.sync_copy(sidx_hbm.at[pl.ds(sb, sw)], sidx_v)
        pltpu.sync_copy(tidx_hbm.at[pl.ds(tb, tw)], tidx_v)
        cp_s = pltpu.async_copy(wtab_hbm.at[sidx_v], srows_v, sem_s)
        cp_t = pltpu.async_copy(ctab_hbm.at[tidx_v], trows_v, sem_t)
        cp_s.wait()
        pltpu.sync_copy(srows_v, xsrc_hbm.at[pl.ds(sb, sw)])
        cp_t.wait()
        pltpu.sync_copy(trows_v, xtgt_hbm.at[pl.ds(tb, tw)])

    return gather


def _scan_body(
    xsrc_ref, xtgt_ref, wxf_ref, wxb_ref, whf_ref, whb_ref, wxd_ref, whd_ref,
    bf_ref, bb_ref, bd_ref,
    mem_ref, hs_ref,
    xgf_ref, xgb_ref, xgd_ref, hf_ref, hb_ref, h_ref,
):
    xs = xsrc_ref[:].astype(jnp.bfloat16)
    xgf_ref[:] = jnp.dot(xs, wxf_ref[:], preferred_element_type=jnp.float32)
    xgb_ref[:] = jnp.dot(xs, wxb_ref[:], preferred_element_type=jnp.float32)
    xgd_ref[:] = jnp.dot(
        xtgt_ref[:].astype(jnp.bfloat16), wxd_ref[:],
        preferred_element_type=jnp.float32,
    )
    hf_ref[:] = jnp.zeros((B, U), jnp.float32)
    hb_ref[:] = jnp.zeros((B, U), jnp.float32)

    def enc_step(s):
        sp = S - 1 - s
        hf = hf_ref[:]
        hb = hb_ref[:]
        hgf = jnp.dot(
            hf.astype(jnp.bfloat16), whf_ref[:], preferred_element_type=jnp.float32
        )
        hgb = jnp.dot(
            hb.astype(jnp.bfloat16), whb_ref[:], preferred_element_type=jnp.float32
        )
        xgf = xgf_ref[pl.ds(s * B, B), :]
        xgb = xgb_ref[pl.ds(sp * B, B), :]
        bf = bf_ref[:]
        bb = bb_ref[:]
        zf = jax.nn.sigmoid(xgf[:, :U] + hgf[:, :U] + bf[:, :U])
        rf = jax.nn.sigmoid(xgf[:, U : 2 * U] + hgf[:, U : 2 * U] + bf[:, U : 2 * U])
        nf = jnp.tanh(xgf[:, 2 * U :] + rf * (hgf[:, 2 * U :] + bf[:, 2 * U :]))
        hfn = (1.0 - zf) * nf + zf * hf
        zb = jax.nn.sigmoid(xgb[:, :U] + hgb[:, :U] + bb[:, :U])
        rb = jax.nn.sigmoid(xgb[:, U : 2 * U] + hgb[:, U : 2 * U] + bb[:, U : 2 * U])
        nb = jnp.tanh(xgb[:, 2 * U :] + rb * (hgb[:, 2 * U :] + bb[:, 2 * U :]))
        hbn = (1.0 - zb) * nb + zb * hb
        hf_ref[:] = hfn
        hb_ref[:] = hbn
        mem_ref[pl.ds(s, 1), :, :U] = hfn[None]
        mem_ref[pl.ds(sp, 1), :, U:] = hbn[None]

    def enc_step2(i, _):
        enc_step(2 * i)
        enc_step(2 * i + 1)
        return 0

    lax.fori_loop(0, S // 2, enc_step2, 0)
    h_ref[:, :U] = hf_ref[:]
    h_ref[:, U:] = hb_ref[:]

    def dec_step(t):
        h = h_ref[:]
        hb16 = h.astype(jnp.bfloat16)
        hg_zr = jnp.dot(
            hb16, whd_ref[:, : 2 * M], preferred_element_type=jnp.float32
        )
        hg_n = jnp.dot(
            hb16, whd_ref[:, 2 * M :], preferred_element_type=jnp.float32
        )
        xg = xgd_ref[pl.ds(t * B, B), :]
        bia = bd_ref[:]
        z = jax.nn.sigmoid(xg[:, :M] + hg_zr[:, :M] + bia[:, :M])
        r = jax.nn.sigmoid(xg[:, M : 2 * M] + hg_zr[:, M:] + bia[:, M : 2 * M])
        n = jnp.tanh(xg[:, 2 * M :] + r * (hg_n + bia[:, 2 * M :]))
        hn = (1.0 - z) * n + z * h
        h_ref[:] = hn
        hs_ref[pl.ds(t, 1)] = hn[None]

    def dec_step2(i, _):
        dec_step(2 * i)
        dec_step(2 * i + 1)
        return 0

    lax.fori_loop(0, T // 2, dec_step2, 0)


def _scans(xsrc, xtgt, wx_f, wx_b, wh_f, wh_b, wx_d, wh_d, b_f2, b_b2, b_d2):
    return pl.pallas_call(
        _scan_body,
        out_shape=(
            jax.ShapeDtypeStruct((S, B, M), jnp.float32),
            jax.ShapeDtypeStruct((T, B, M), jnp.float32),
        ),
        scratch_shapes=[
            pltpu.VMEM((S * B, 3 * U), jnp.float32),
            pltpu.VMEM((S * B, 3 * U), jnp.float32),
            pltpu.VMEM((T * B, 3 * M), jnp.float32),
            pltpu.VMEM((B, U), jnp.float32),
            pltpu.VMEM((B, U), jnp.float32),
            pltpu.VMEM((B, M), jnp.float32),
        ],
    )(xsrc, xtgt, wx_f, wx_b, wh_f, wh_b, wx_d, wh_d, b_f2, b_b2, b_d2)


_VBLK = 4096


def _attn_logits_body(hs_ref, mem_ref, wc_ref, wo_ref, bo_ref, out_ref, comb_ref):
    @pl.when(pl.program_id(0) == 0)
    def _():
        wc_h = wc_ref[:M, :]
        wc_c = wc_ref[M:, :]
        for b in range(B):
            hb = hs_ref[b]
            mb = mem_ref[b]
            hb16 = hb.astype(jnp.bfloat16)
            mb16 = mb.astype(jnp.bfloat16)
            scores = lax.dot_general(
                hb16, mb16, (((1,), (1,)), ((), ())),
                preferred_element_type=jnp.float32,
            )
            mx = jnp.max(scores, axis=-1, keepdims=True)
            e = jnp.exp(scores - mx)
            attn = (e / jnp.sum(e, axis=-1, keepdims=True)).astype(jnp.bfloat16)
            ctx = jnp.dot(attn, mb16, preferred_element_type=jnp.float32)
            comb = jnp.tanh(
                jnp.dot(hb16, wc_h, preferred_element_type=jnp.float32)
                + jnp.dot(ctx.astype(jnp.bfloat16), wc_c,
                          preferred_element_type=jnp.float32)
            )
            comb_ref[b * T : (b + 1) * T, :] = comb.astype(jnp.bfloat16)

    out_ref[:] = (
        jnp.dot(comb_ref[:], wo_ref[:], preferred_element_type=jnp.float32)
        + bo_ref[:]
    )


def _attn_logits(hs_btm, mem_bsm, w_c, w_o, b_o2):
    nblk = pl.cdiv(V, _VBLK)
    return pl.pallas_call(
        _attn_logits_body,
        grid=(nblk,),
        in_specs=[
            pl.BlockSpec((B, T, M), lambda j: (0, 0, 0)),
            pl.BlockSpec((B, S, M), lambda j: (0, 0, 0)),
            pl.BlockSpec((2 * M, M), lambda j: (0, 0)),
            pl.BlockSpec((M, _VBLK), lambda j: (0, j)),
            pl.BlockSpec((1, _VBLK), lambda j: (0, j)),
        ],
        out_specs=pl.BlockSpec((B * T, _VBLK), lambda j: (0, j)),
        out_shape=jax.ShapeDtypeStruct((B * T, V), jnp.float32),
        scratch_shapes=[pltpu.VMEM((B * T, M), jnp.bfloat16)],
    )(hs_btm, mem_bsm, w_c, w_o, b_o2)


def kernel(word_embed, code_embed, Wx_f, Wh_f, b_f, Wx_b, Wh_b, b_b,
           Wx_d, Wh_d, b_d, W_c, W_o, b_o, src_tokens, tgt_tokens):
    bf16 = jnp.bfloat16
    src_idx = src_tokens.T.reshape(-1)
    tgt_idx = tgt_tokens.T.reshape(-1)
    xsrc, xtgt = _make_sc_gather()(word_embed, src_idx, code_embed, tgt_idx)

    mem_sbm, hs_tbm = _scans(
        xsrc, xtgt,
        Wx_f.astype(bf16), Wx_b.astype(bf16),
        Wh_f.astype(bf16), Wh_b.astype(bf16),
        Wx_d.astype(bf16), Wh_d.astype(bf16),
        b_f.reshape(1, 3 * U), b_b.reshape(1, 3 * U), b_d.reshape(1, 3 * M),
    )

    mem_bsm = mem_sbm.transpose(1, 0, 2)
    hs_btm = hs_tbm.transpose(1, 0, 2)
    logits = _attn_logits(
        hs_btm, mem_bsm, W_c.astype(bf16), W_o.astype(bf16), b_o.reshape(1, V)
    )
    return logits.reshape(B, T, V)


# speedup vs baseline: 1.0380x; 1.0380x over previous
import functools

import jax
import jax.numpy as jnp
from jax import lax
from jax.experimental import pallas as pl
from jax.experimental.pallas import tpu as pltpu
from jax.experimental.pallas import tpu_sc as plsc

B = 8
S = 128
T = 64
U = 512
M = 2 * U
V = 8020


@functools.lru_cache(maxsize=None)
def _make_sc_gather():
    info = plsc.get_sparse_core_info()
    nw = info.num_cores * info.num_subcores
    sw = S * B // nw
    tw = T * B // nw
    mesh = plsc.VectorSubcoreMesh(core_axis_name="c", subcore_axis_name="s")

    @functools.partial(
        pl.kernel,
        mesh=mesh,
        out_type=(
            jax.ShapeDtypeStruct((S * B, U), jnp.float32),
            jax.ShapeDtypeStruct((T * B, U), jnp.float32),
        ),
        scratch_types=[
            pltpu.VMEM((sw,), jnp.int32),
            pltpu.VMEM((sw, U), jnp.float32),
            pltpu.VMEM((tw,), jnp.int32),
            pltpu.VMEM((tw, U), jnp.float32),
            pltpu.SemaphoreType.DMA,
            pltpu.SemaphoreType.DMA,
        ],
    )
    def gather(wtab_hbm, sidx_hbm, ctab_hbm, tidx_hbm, xsrc_hbm, xtgt_hbm,
               sidx_v, srows_v, tidx_v, trows_v, sem_s, sem_t):
        wid = lax.axis_index("s") * info.num_cores + lax.axis_index("c")
        sb = wid * sw
        tb = wid * tw
        pltpu.sync_copy(sidx_hbm.at[pl.ds(sb, sw)], sidx_v)
        pltpu.sync_copy(tidx_hbm.at[pl.ds(tb, tw)], tidx_v)
        cp_s = pltpu.async_copy(wtab_hbm.at[sidx_v], srows_v, sem_s)
        cp_t = pltpu.async_copy(ctab_hbm.at[tidx_v], trows_v, sem_t)
        cp_s.wait()
        pltpu.sync_copy(srows_v, xsrc_hbm.at[pl.ds(sb, sw)])
        cp_t.wait()
        pltpu.sync_copy(trows_v, xtgt_hbm.at[pl.ds(tb, tw)])

    return gather


_VBLK = 1024


def _mega_body(
    xsrc_ref, xtgt_ref, wxf_ref, wxb_ref, whf_ref, whb_ref, wxd_ref, whd_ref,
    bf_ref, bb_ref, bd_ref, wc_ref, wo_ref, bo_ref,
    out_ref,
    xgf_ref, xgb_ref, xgd_ref, hf_ref, hb_ref, h_ref, mem_ref, hs_ref, comb_ref,
):
    @pl.when(pl.program_id(0) == 0)
    def _():
        xs = xsrc_ref[:].astype(jnp.bfloat16)
        xgf_ref[:] = jnp.dot(
            xs, wxf_ref[:], preferred_element_type=jnp.float32
        ).astype(jnp.bfloat16)
        xgb_ref[:] = jnp.dot(
            xs, wxb_ref[:], preferred_element_type=jnp.float32
        ).astype(jnp.bfloat16)
        xgd_ref[:] = jnp.dot(
            xtgt_ref[:].astype(jnp.bfloat16), wxd_ref[:],
            preferred_element_type=jnp.float32,
        ).astype(jnp.bfloat16)
        hf_ref[:] = jnp.zeros((B, U), jnp.float32)
        hb_ref[:] = jnp.zeros((B, U), jnp.float32)

        def enc_step(s, sp, xgf, xgb):
            hf = hf_ref[:]
            hb = hb_ref[:]
            hgf = jnp.dot(
                hf.astype(jnp.bfloat16), whf_ref[:],
                preferred_element_type=jnp.float32,
            )
            hgb = jnp.dot(
                hb.astype(jnp.bfloat16), whb_ref[:],
                preferred_element_type=jnp.float32,
            )
            bf = bf_ref[:]
            bb = bb_ref[:]
            zf = jax.nn.sigmoid(xgf[:, :U] + hgf[:, :U] + bf[:, :U])
            rf = jax.nn.sigmoid(
                xgf[:, U : 2 * U] + hgf[:, U : 2 * U] + bf[:, U : 2 * U]
            )
            nf = jnp.tanh(xgf[:, 2 * U :] + rf * (hgf[:, 2 * U :] + bf[:, 2 * U :]))
            hfn = (1.0 - zf) * nf + zf * hf
            zb = jax.nn.sigmoid(xgb[:, :U] + hgb[:, :U] + bb[:, :U])
            rb = jax.nn.sigmoid(
                xgb[:, U : 2 * U] + hgb[:, U : 2 * U] + bb[:, U : 2 * U]
            )
            nb = jnp.tanh(xgb[:, 2 * U :] + rb * (hgb[:, 2 * U :] + bb[:, 2 * U :]))
            hbn = (1.0 - zb) * nb + zb * hb
            hf_ref[:] = hfn
            hb_ref[:] = hbn
            mem_ref[:, pl.ds(s, 1), :U] = hfn[:, None, :]
            mem_ref[:, pl.ds(sp, 1), U:] = hbn[:, None, :]

        def enc_step2(i, _):
            xgf2 = xgf_ref[pl.ds(i * 2 * B, 2 * B), :].astype(jnp.float32)
            xgb2 = xgb_ref[pl.ds((S // 2 - 1 - i) * 2 * B, 2 * B), :].astype(
                jnp.float32
            )
            enc_step(2 * i, S - 1 - 2 * i, xgf2[:B], xgb2[B:])
            enc_step(2 * i + 1, S - 2 - 2 * i, xgf2[B:], xgb2[:B])
            return 0

        lax.fori_loop(0, S // 2, enc_step2, 0)
        h_ref[:, :U] = hf_ref[:]
        h_ref[:, U:] = hb_ref[:]

        def dec_step(t, xg):
            h = h_ref[:]
            hb16 = h.astype(jnp.bfloat16)
            hg_zr = jnp.dot(
                hb16, whd_ref[:, : 2 * M], preferred_element_type=jnp.float32
            )
            hg_n = jnp.dot(
                hb16, whd_ref[:, 2 * M :], preferred_element_type=jnp.float32
            )
            bia = bd_ref[:]
            z = jax.nn.sigmoid(xg[:, :M] + hg_zr[:, :M] + bia[:, :M])
            r = jax.nn.sigmoid(xg[:, M : 2 * M] + hg_zr[:, M:] + bia[:, M : 2 * M])
            n = jnp.tanh(xg[:, 2 * M :] + r * (hg_n + bia[:, 2 * M :]))
            hn = (1.0 - z) * n + z * h
            h_ref[:] = hn
            hs_ref[:, pl.ds(t, 1), :] = hn[:, None, :]

        def dec_step2(i, _):
            xg2 = xgd_ref[pl.ds(i * 2 * B, 2 * B), :].astype(jnp.float32)
            dec_step(2 * i, xg2[:B])
            dec_step(2 * i + 1, xg2[B:])
            return 0

        lax.fori_loop(0, T // 2, dec_step2, 0)

        wc_h = wc_ref[:M, :]
        wc_c = wc_ref[M:, :]
        for b in range(B):
            hsb = hs_ref[b]
            mb = mem_ref[b]
            hb16 = hsb.astype(jnp.bfloat16)
            mb16 = mb.astype(jnp.bfloat16)
            scores = lax.dot_general(
                hb16, mb16, (((1,), (1,)), ((), ())),
                preferred_element_type=jnp.float32,
            )
            mx = jnp.max(scores, axis=-1, keepdims=True)
            e = jnp.exp(scores - mx)
            attn = (e / jnp.sum(e, axis=-1, keepdims=True)).astype(jnp.bfloat16)
            ctx = jnp.dot(attn, mb16, preferred_element_type=jnp.float32)
            comb = jnp.tanh(
                jnp.dot(hb16, wc_h, preferred_element_type=jnp.float32)
                + jnp.dot(ctx.astype(jnp.bfloat16), wc_c,
                          preferred_element_type=jnp.float32)
            )
            comb_ref[b * T : (b + 1) * T, :] = comb.astype(jnp.bfloat16)

    out_ref[:] = (
        jnp.dot(comb_ref[:], wo_ref[:], preferred_element_type=jnp.float32)
        + bo_ref[:]
    )


def _mega(xsrc, xtgt, wx_f, wx_b, wh_f, wh_b, wx_d, wh_d, b_f2, b_b2, b_d2,
          w_c, w_o, b_o2):
    nblk = pl.cdiv(V, _VBLK)
    full = lambda j: (0, 0)
    return pl.pallas_call(
        _mega_body,
        grid=(nblk,),
        in_specs=[
            pl.BlockSpec((S * B, U), full),
            pl.BlockSpec((T * B, U), full),
            pl.BlockSpec((U, 3 * U), full),
            pl.BlockSpec((U, 3 * U), full),
            pl.BlockSpec((U, 3 * U), full),
            pl.BlockSpec((U, 3 * U), full),
            pl.BlockSpec((U, 3 * M), full),
            pl.BlockSpec((M, 3 * M), full),
            pl.BlockSpec((1, 3 * U), full),
            pl.BlockSpec((1, 3 * U), full),
            pl.BlockSpec((1, 3 * M), full),
            pl.BlockSpec((2 * M, M), full),
            pl.BlockSpec((M, _VBLK), lambda j: (0, j)),
            pl.BlockSpec((1, _VBLK), lambda j: (0, j)),
        ],
        out_specs=pl.BlockSpec((B * T, _VBLK), lambda j: (0, j)),
        out_shape=jax.ShapeDtypeStruct((B * T, V), jnp.float32),
        scratch_shapes=[
            pltpu.VMEM((S * B, 3 * U), jnp.bfloat16),
            pltpu.VMEM((S * B, 3 * U), jnp.bfloat16),
            pltpu.VMEM((T * B, 3 * M), jnp.bfloat16),
            pltpu.VMEM((B, U), jnp.float32),
            pltpu.VMEM((B, U), jnp.float32),
            pltpu.VMEM((B, M), jnp.float32),
            pltpu.VMEM((B, S, M), jnp.float32),
            pltpu.VMEM((B, T, M), jnp.float32),
            pltpu.VMEM((B * T, M), jnp.bfloat16),
        ],
    )(xsrc, xtgt, wx_f, wx_b, wh_f, wh_b, wx_d, wh_d, b_f2, b_b2, b_d2,
      w_c, w_o, b_o2)


def kernel(word_embed, code_embed, Wx_f, Wh_f, b_f, Wx_b, Wh_b, b_b,
           Wx_d, Wh_d, b_d, W_c, W_o, b_o, src_tokens, tgt_tokens):
    bf16 = jnp.bfloat16
    src_idx = src_tokens.T.reshape(-1)
    tgt_idx = tgt_tokens.T.reshape(-1)
    xsrc, xtgt = _make_sc_gather()(word_embed, src_idx, code_embed, tgt_idx)

    logits = _mega(
        xsrc, xtgt,
        Wx_f.astype(bf16), Wx_b.astype(bf16),
        Wh_f.astype(bf16), Wh_b.astype(bf16),
        Wx_d.astype(bf16), Wh_d.astype(bf16),
        b_f.reshape(1, 3 * U), b_b.reshape(1, 3 * U), b_d.reshape(1, 3 * M),
        W_c.astype(bf16), W_o.astype(bf16), b_o.reshape(1, V),
    )
    return logits.reshape(B, T, V)


# speedup vs baseline: 1.0755x; 1.0361x over previous
import functools

import jax
import jax.numpy as jnp
from jax import lax
from jax.experimental import pallas as pl
from jax.experimental.pallas import tpu as pltpu
from jax.experimental.pallas import tpu_sc as plsc

B = 8
S = 128
T = 64
U = 512
M = 2 * U
V = 8020


@functools.lru_cache(maxsize=None)
def _make_sc_gather():
    info = plsc.get_sparse_core_info()
    nw = info.num_cores * info.num_subcores
    sw = S * B // nw
    tw = T * B // nw
    mesh = plsc.VectorSubcoreMesh(core_axis_name="c", subcore_axis_name="s")

    @functools.partial(
        pl.kernel,
        mesh=mesh,
        out_type=(
            jax.ShapeDtypeStruct((S * B, U), jnp.float32),
            jax.ShapeDtypeStruct((T * B, U), jnp.float32),
        ),
        scratch_types=[
            pltpu.VMEM((sw,), jnp.int32),
            pltpu.VMEM((sw, U), jnp.float32),
            pltpu.VMEM((tw,), jnp.int32),
            pltpu.VMEM((tw, U), jnp.float32),
            pltpu.SemaphoreType.DMA,
            pltpu.SemaphoreType.DMA,
        ],
    )
    def gather(wtab_hbm, sidx_hbm, ctab_hbm, tidx_hbm, xsrc_hbm, xtgt_hbm,
               sidx_v, srows_v, tidx_v, trows_v, sem_s, sem_t):
        wid = lax.axis_index("s") * info.num_cores + lax.axis_index("c")
        sb = wid * sw
        tb = wid * tw
        pltpu.sync_copy(sidx_hbm.at[pl.ds(sb, sw)], sidx_v)
        pltpu.sync_copy(tidx_hbm.at[pl.ds(tb, tw)], tidx_v)
        cp_s = pltpu.async_copy(wtab_hbm.at[sidx_v], srows_v, sem_s)
        cp_t = pltpu.async_copy(ctab_hbm.at[tidx_v], trows_v, sem_t)
        cp_s.wait()
        pltpu.sync_copy(srows_v, xsrc_hbm.at[pl.ds(sb, sw)])
        cp_t.wait()
        pltpu.sync_copy(trows_v, xtgt_hbm.at[pl.ds(tb, tw)])

    return gather


def _xg_body(xsrc_ref, xtgt_ref, wxf_ref, wxb_ref, wxd_ref,
             xgf_ref, xgb_ref, xgd_ref):
    xs = xsrc_ref[:].astype(jnp.bfloat16)
    xgf_ref[:] = jnp.dot(
        xs, wxf_ref[:].astype(jnp.bfloat16), preferred_element_type=jnp.float32
    ).astype(jnp.bfloat16)
    xgb_ref[:] = jnp.dot(
        xs, wxb_ref[:].astype(jnp.bfloat16), preferred_element_type=jnp.float32
    ).astype(jnp.bfloat16)
    xgd_ref[:] = jnp.dot(
        xtgt_ref[:].astype(jnp.bfloat16), wxd_ref[:].astype(jnp.bfloat16),
        preferred_element_type=jnp.float32,
    ).astype(jnp.bfloat16)


def _xg_proj(xsrc, xtgt, wx_f, wx_b, wx_d):
    return pl.pallas_call(
        _xg_body,
        out_shape=(
            jax.ShapeDtypeStruct((S * B, 3 * U), jnp.bfloat16),
            jax.ShapeDtypeStruct((S * B, 3 * U), jnp.bfloat16),
            jax.ShapeDtypeStruct((T * B, 3 * M), jnp.bfloat16),
        ),
    )(xsrc, xtgt, wx_f, wx_b, wx_d)


_VBLK = 512


def _mega_body(
    xgf_ref, xgb_ref, xgd_ref, whf_ref, whb_ref, whd_ref,
    bf_ref, bb_ref, bd_ref, wc_ref, wo_ref, bo_ref,
    out_ref,
    hf_ref, hb_ref, h_ref, mem_ref, hs_ref, comb_ref,
):
    @pl.when(pl.program_id(0) == 0)
    def _():
        hf_ref[:] = jnp.zeros((B, U), jnp.float32)
        hb_ref[:] = jnp.zeros((B, U), jnp.float32)

        def enc_step(s, sp, xgf, xgb):
            hf = hf_ref[:]
            hb = hb_ref[:]
            hgf = jnp.dot(hf, whf_ref[:], preferred_element_type=jnp.float32)
            hgb = jnp.dot(hb, whb_ref[:], preferred_element_type=jnp.float32)
            bf = bf_ref[:]
            bb = bb_ref[:]
            zf = jax.nn.sigmoid(xgf[:, :U] + hgf[:, :U] + bf[:, :U])
            rf = jax.nn.sigmoid(
                xgf[:, U : 2 * U] + hgf[:, U : 2 * U] + bf[:, U : 2 * U]
            )
            nf = jnp.tanh(xgf[:, 2 * U :] + rf * (hgf[:, 2 * U :] + bf[:, 2 * U :]))
            hfn = (1.0 - zf) * nf + zf * hf
            zb = jax.nn.sigmoid(xgb[:, :U] + hgb[:, :U] + bb[:, :U])
            rb = jax.nn.sigmoid(
                xgb[:, U : 2 * U] + hgb[:, U : 2 * U] + bb[:, U : 2 * U]
            )
            nb = jnp.tanh(xgb[:, 2 * U :] + rb * (hgb[:, 2 * U :] + bb[:, 2 * U :]))
            hbn = (1.0 - zb) * nb + zb * hb
            hf_ref[:] = hfn
            hb_ref[:] = hbn
            mem_ref[:, pl.ds(s, 1), :U] = hfn[:, None, :]
            mem_ref[:, pl.ds(sp, 1), U:] = hbn[:, None, :]

        def enc_step2(i, _):
            xgf2 = xgf_ref[pl.ds(i * 2 * B, 2 * B), :].astype(jnp.float32)
            xgb2 = xgb_ref[pl.ds((S // 2 - 1 - i) * 2 * B, 2 * B), :].astype(
                jnp.float32
            )
            enc_step(2 * i, S - 1 - 2 * i, xgf2[:B], xgb2[B:])
            enc_step(2 * i + 1, S - 2 - 2 * i, xgf2[B:], xgb2[:B])
            return 0

        lax.fori_loop(0, S // 2, enc_step2, 0)
        h_ref[:, :U] = hf_ref[:]
        h_ref[:, U:] = hb_ref[:]

        def dec_step(t, xg):
            h = h_ref[:]
            hg_zr = jnp.dot(
                h, whd_ref[:, : 2 * M], preferred_element_type=jnp.float32
            )
            hg_n = jnp.dot(
                h, whd_ref[:, 2 * M :], preferred_element_type=jnp.float32
            )
            bia = bd_ref[:]
            z = jax.nn.sigmoid(xg[:, :M] + hg_zr[:, :M] + bia[:, :M])
            r = jax.nn.sigmoid(xg[:, M : 2 * M] + hg_zr[:, M:] + bia[:, M : 2 * M])
            n = jnp.tanh(xg[:, 2 * M :] + r * (hg_n + bia[:, 2 * M :]))
            hn = (1.0 - z) * n + z * h
            h_ref[:] = hn
            hs_ref[:, pl.ds(t, 1), :] = hn[:, None, :]

        def dec_step2(i, _):
            xg2 = xgd_ref[pl.ds(i * 2 * B, 2 * B), :].astype(jnp.float32)
            dec_step(2 * i, xg2[:B])
            dec_step(2 * i + 1, xg2[B:])
            return 0

        lax.fori_loop(0, T // 2, dec_step2, 0)

        wc_h = wc_ref[:M, :]
        wc_c = wc_ref[M:, :]
        for b in range(B):
            hsb = hs_ref[b]
            mb = mem_ref[b]
            scores = lax.dot_general(
                hsb, mb, (((1,), (1,)), ((), ())),
                preferred_element_type=jnp.float32,
            )
            mx = jnp.max(scores, axis=-1, keepdims=True)
            e = jnp.exp(scores - mx)
            attn = e / jnp.sum(e, axis=-1, keepdims=True)
            ctx = jnp.dot(attn, mb, preferred_element_type=jnp.float32)
            comb = jnp.tanh(
                jnp.dot(hsb, wc_h, preferred_element_type=jnp.float32)
                + jnp.dot(ctx, wc_c, preferred_element_type=jnp.float32)
            )
            comb_ref[b * T : (b + 1) * T, :] = comb.astype(jnp.bfloat16)

    out_ref[:] = (
        jnp.dot(
            comb_ref[:], wo_ref[:].astype(jnp.bfloat16),
            preferred_element_type=jnp.float32,
        )
        + bo_ref[:]
    )


def _mega(xgf, xgb, xgd, wh_f, wh_b, wh_d, b_f2, b_b2, b_d2, w_c, w_o, b_o2):
    nblk = pl.cdiv(V, _VBLK)
    full = lambda j: (0, 0)
    return pl.pallas_call(
        _mega_body,
        grid=(nblk,),
        in_specs=[
            pl.BlockSpec((S * B, 3 * U), full),
            pl.BlockSpec((S * B, 3 * U), full),
            pl.BlockSpec((T * B, 3 * M), full),
            pl.BlockSpec((U, 3 * U), full),
            pl.BlockSpec((U, 3 * U), full),
            pl.BlockSpec((M, 3 * M), full),
            pl.BlockSpec((1, 3 * U), full),
            pl.BlockSpec((1, 3 * U), full),
            pl.BlockSpec((1, 3 * M), full),
            pl.BlockSpec((2 * M, M), full),
            pl.BlockSpec((M, _VBLK), lambda j: (0, j)),
            pl.BlockSpec((1, _VBLK), lambda j: (0, j)),
        ],
        out_specs=pl.BlockSpec((B * T, _VBLK), lambda j: (0, j)),
        out_shape=jax.ShapeDtypeStruct((B * T, V), jnp.float32),
        scratch_shapes=[
            pltpu.VMEM((B, U), jnp.float32),
            pltpu.VMEM((B, U), jnp.float32),
            pltpu.VMEM((B, M), jnp.float32),
            pltpu.VMEM((B, S, M), jnp.float32),
            pltpu.VMEM((B, T, M), jnp.float32),
            pltpu.VMEM((B * T, M), jnp.bfloat16),
        ],
    )(xgf, xgb, xgd, wh_f, wh_b, wh_d, b_f2, b_b2, b_d2, w_c, w_o, b_o2)


def kernel(word_embed, code_embed, Wx_f, Wh_f, b_f, Wx_b, Wh_b, b_b,
           Wx_d, Wh_d, b_d, W_c, W_o, b_o, src_tokens, tgt_tokens):
    src_idx = src_tokens.T.reshape(-1)
    tgt_idx = tgt_tokens.T.reshape(-1)
    xsrc, xtgt = _make_sc_gather()(word_embed, src_idx, code_embed, tgt_idx)

    xgf, xgb, xgd = _xg_proj(xsrc, xtgt, Wx_f, Wx_b, Wx_d)
    logits = _mega(
        xgf, xgb, xgd, Wh_f, Wh_b, Wh_d,
        b_f.reshape(1, 3 * U), b_b.reshape(1, 3 * U), b_d.reshape(1, 3 * M),
        W_c.astype(jnp.bfloat16), W_o, b_o.reshape(1, V),
    )
    return logits.reshape(B, T, V)


# speedup vs baseline: 1.1054x; 1.0278x over previous
import functools

import jax
import jax.numpy as jnp
from jax import lax
from jax.experimental import pallas as pl
from jax.experimental.pallas import tpu as pltpu
from jax.experimental.pallas import tpu_sc as plsc

B = 8
S = 128
T = 64
U = 512
M = 2 * U
V = 8020


@functools.lru_cache(maxsize=None)
def _make_sc_gather():
    info = plsc.get_sparse_core_info()
    nw = info.num_cores * info.num_subcores
    sw = S * B // nw
    tw = T * B // nw
    mesh = plsc.VectorSubcoreMesh(core_axis_name="c", subcore_axis_name="s")

    @functools.partial(
        pl.kernel,
        mesh=mesh,
        out_type=(
            jax.ShapeDtypeStruct((S * B, U), jnp.float32),
            jax.ShapeDtypeStruct((T * B, U), jnp.float32),
        ),
        scratch_types=[
            pltpu.VMEM((sw,), jnp.int32),
            pltpu.VMEM((sw, U), jnp.float32),
            pltpu.VMEM((tw,), jnp.int32),
            pltpu.VMEM((tw, U), jnp.float32),
            pltpu.SemaphoreType.DMA,
            pltpu.SemaphoreType.DMA,
        ],
    )
    def gather(wtab_hbm, sidx_hbm, ctab_hbm, tidx_hbm, xsrc_hbm, xtgt_hbm,
               sidx_v, srows_v, tidx_v, trows_v, sem_s, sem_t):
        wid = lax.axis_index("s") * info.num_cores + lax.axis_index("c")
        sb = wid * sw
        tb = wid * tw
        pltpu.sync_copy(sidx_hbm.at[pl.ds(sb, sw)], sidx_v)
        pltpu.sync_copy(tidx_hbm.at[pl.ds(tb, tw)], tidx_v)
        cp_s = pltpu.async_copy(wtab_hbm.at[sidx_v], srows_v, sem_s)
        cp_t = pltpu.async_copy(ctab_hbm.at[tidx_v], trows_v, sem_t)
        cp_s.wait()
        pltpu.sync_copy(srows_v, xsrc_hbm.at[pl.ds(sb, sw)])
        cp_t.wait()
        pltpu.sync_copy(trows_v, xtgt_hbm.at[pl.ds(tb, tw)])

    return gather


def _xg_body(xsrc_ref, xtgt_ref, wxf_ref, wxb_ref, wxd_ref,
             xgf_ref, xgb_ref, xgd_ref):
    xs = xsrc_ref[:].astype(jnp.bfloat16)
    xgf_ref[:] = jnp.dot(
        xs, wxf_ref[:].astype(jnp.bfloat16), preferred_element_type=jnp.float32
    ).astype(jnp.bfloat16)
    xgb_ref[:] = jnp.dot(
        xs, wxb_ref[:].astype(jnp.bfloat16), preferred_element_type=jnp.float32
    ).astype(jnp.bfloat16)
    xgd_ref[:] = jnp.dot(
        xtgt_ref[:].astype(jnp.bfloat16), wxd_ref[:].astype(jnp.bfloat16),
        preferred_element_type=jnp.float32,
    ).astype(jnp.bfloat16)


def _xg_proj(xsrc, xtgt, wx_f, wx_b, wx_d):
    return pl.pallas_call(
        _xg_body,
        out_shape=(
            jax.ShapeDtypeStruct((S * B, 3 * U), jnp.bfloat16),
            jax.ShapeDtypeStruct((S * B, 3 * U), jnp.bfloat16),
            jax.ShapeDtypeStruct((T * B, 3 * M), jnp.bfloat16),
        ),
    )(xsrc, xtgt, wx_f, wx_b, wx_d)


_VBLK = 512


def _mega_body(
    xgf_ref, xgb_ref, xgd_ref, whf_ref, whb_ref, whd_ref,
    bf_ref, bb_ref, bd_ref, wc_ref, wo_ref, bo_ref,
    out_ref,
    hf_ref, hb_ref, h_ref, mem_ref, hs_ref, comb_ref,
):
    @pl.when(pl.program_id(0) == 0)
    def _():
        hf_ref[:] = jnp.zeros((B, U), jnp.float32)
        hb_ref[:] = jnp.zeros((B, U), jnp.float32)

        def enc_step(s, sp, xgf, xgb):
            hf = hf_ref[:]
            hb = hb_ref[:]
            hgf = jnp.dot(hf, whf_ref[:], preferred_element_type=jnp.float32)
            hgb = jnp.dot(hb, whb_ref[:], preferred_element_type=jnp.float32)
            bf = bf_ref[:]
            bb = bb_ref[:]
            zf = jax.nn.sigmoid(xgf[:, :U] + hgf[:, :U] + bf[:, :U])
            rf = jax.nn.sigmoid(
                xgf[:, U : 2 * U] + hgf[:, U : 2 * U] + bf[:, U : 2 * U]
            )
            nf = jnp.tanh(xgf[:, 2 * U :] + rf * (hgf[:, 2 * U :] + bf[:, 2 * U :]))
            hfn = (1.0 - zf) * nf + zf * hf
            zb = jax.nn.sigmoid(xgb[:, :U] + hgb[:, :U] + bb[:, :U])
            rb = jax.nn.sigmoid(
                xgb[:, U : 2 * U] + hgb[:, U : 2 * U] + bb[:, U : 2 * U]
            )
            nb = jnp.tanh(xgb[:, 2 * U :] + rb * (hgb[:, 2 * U :] + bb[:, 2 * U :]))
            hbn = (1.0 - zb) * nb + zb * hb
            hf_ref[:] = hfn
            hb_ref[:] = hbn
            mem_ref[:, pl.ds(s, 1), :U] = hfn[:, None, :]
            mem_ref[:, pl.ds(sp, 1), U:] = hbn[:, None, :]

        def enc_step4(i, _):
            xgf4 = xgf_ref[pl.ds(i * 4 * B, 4 * B), :].astype(jnp.float32)
            xgb4 = xgb_ref[pl.ds((S // 4 - 1 - i) * 4 * B, 4 * B), :].astype(
                jnp.float32
            )
            for k in range(4):
                enc_step(
                    4 * i + k,
                    S - 1 - 4 * i - k,
                    xgf4[k * B : (k + 1) * B],
                    xgb4[(3 - k) * B : (4 - k) * B],
                )
            return 0

        lax.fori_loop(0, S // 4, enc_step4, 0)
        h_ref[:, :U] = hf_ref[:]
        h_ref[:, U:] = hb_ref[:]

        def dec_step(t, xg):
            h = h_ref[:]
            hg_zr = jnp.dot(
                h, whd_ref[:, : 2 * M], preferred_element_type=jnp.float32
            )
            hg_n = jnp.dot(
                h, whd_ref[:, 2 * M :], preferred_element_type=jnp.float32
            )
            bia = bd_ref[:]
            z = jax.nn.sigmoid(xg[:, :M] + hg_zr[:, :M] + bia[:, :M])
            r = jax.nn.sigmoid(xg[:, M : 2 * M] + hg_zr[:, M:] + bia[:, M : 2 * M])
            n = jnp.tanh(xg[:, 2 * M :] + r * (hg_n + bia[:, 2 * M :]))
            hn = (1.0 - z) * n + z * h
            h_ref[:] = hn
            hs_ref[:, pl.ds(t, 1), :] = hn[:, None, :]

        def dec_step4(i, _):
            xg4 = xgd_ref[pl.ds(i * 4 * B, 4 * B), :].astype(jnp.float32)
            for k in range(4):
                dec_step(4 * i + k, xg4[k * B : (k + 1) * B])
            return 0

        lax.fori_loop(0, T // 4, dec_step4, 0)

        wc_h = wc_ref[:M, :]
        wc_c = wc_ref[M:, :]
        for b in range(B):
            hsb = hs_ref[b]
            mb = mem_ref[b]
            scores = lax.dot_general(
                hsb, mb, (((1,), (1,)), ((), ())),
                preferred_element_type=jnp.float32,
            )
            mx = jnp.max(scores, axis=-1, keepdims=True)
            e = jnp.exp(scores - mx)
            attn = e / jnp.sum(e, axis=-1, keepdims=True)
            ctx = jnp.dot(attn, mb, preferred_element_type=jnp.float32)
            comb = jnp.tanh(
                jnp.dot(hsb, wc_h, preferred_element_type=jnp.float32)
                + jnp.dot(ctx, wc_c, preferred_element_type=jnp.float32)
            )
            comb_ref[b * T : (b + 1) * T, :] = comb.astype(jnp.bfloat16)

    out_ref[:] = (
        jnp.dot(
            comb_ref[:], wo_ref[:].astype(jnp.bfloat16),
            preferred_element_type=jnp.float32,
        )
        + bo_ref[:]
    )


def _mega(xgf, xgb, xgd, wh_f, wh_b, wh_d, b_f2, b_b2, b_d2, w_c, w_o, b_o2):
    nblk = pl.cdiv(V, _VBLK)
    full = lambda j: (0, 0)
    return pl.pallas_call(
        _mega_body,
        grid=(nblk,),
        in_specs=[
            pl.BlockSpec((S * B, 3 * U), full),
            pl.BlockSpec((S * B, 3 * U), full),
            pl.BlockSpec((T * B, 3 * M), full),
            pl.BlockSpec((U, 3 * U), full),
            pl.BlockSpec((U, 3 * U), full),
            pl.BlockSpec((M, 3 * M), full),
            pl.BlockSpec((1, 3 * U), full),
            pl.BlockSpec((1, 3 * U), full),
            pl.BlockSpec((1, 3 * M), full),
            pl.BlockSpec((2 * M, M), full),
            pl.BlockSpec((M, _VBLK), lambda j: (0, j)),
            pl.BlockSpec((1, _VBLK), lambda j: (0, j)),
        ],
        out_specs=pl.BlockSpec((B * T, _VBLK), lambda j: (0, j)),
        out_shape=jax.ShapeDtypeStruct((B * T, V), jnp.float32),
        scratch_shapes=[
            pltpu.VMEM((B, U), jnp.float32),
            pltpu.VMEM((B, U), jnp.float32),
            pltpu.VMEM((B, M), jnp.float32),
            pltpu.VMEM((B, S, M), jnp.float32),
            pltpu.VMEM((B, T, M), jnp.float32),
            pltpu.VMEM((B * T, M), jnp.bfloat16),
        ],
    )(xgf, xgb, xgd, wh_f, wh_b, wh_d, b_f2, b_b2, b_d2, w_c, w_o, b_o2)


def kernel(word_embed, code_embed, Wx_f, Wh_f, b_f, Wx_b, Wh_b, b_b,
           Wx_d, Wh_d, b_d, W_c, W_o, b_o, src_tokens, tgt_tokens):
    src_idx = src_tokens.T.reshape(-1)
    tgt_idx = tgt_tokens.T.reshape(-1)
    xsrc, xtgt = _make_sc_gather()(word_embed, src_idx, code_embed, tgt_idx)

    xgf, xgb, xgd = _xg_proj(xsrc, xtgt, Wx_f, Wx_b, Wx_d)
    logits = _mega(
        xgf, xgb, xgd, Wh_f, Wh_b, Wh_d,
        b_f.reshape(1, 3 * U), b_b.reshape(1, 3 * U), b_d.reshape(1, 3 * M),
        W_c.astype(jnp.bfloat16), W_o, b_o.reshape(1, V),
    )
    return logits.reshape(B, T, V)


# speedup vs baseline: 1.1054x; 1.0000x over previous
import functools

import jax
import jax.numpy as jnp
from jax import lax
from jax.experimental import pallas as pl
from jax.experimental.pallas import tpu as pltpu
from jax.experimental.pallas import tpu_sc as plsc

B = 8
S = 128
T = 64
U = 512
M = 2 * U
V = 8020


@functools.lru_cache(maxsize=None)
def _make_sc_gather():
    info = plsc.get_sparse_core_info()
    nw = info.num_cores * info.num_subcores
    sw = S * B // nw
    tw = T * B // nw
    mesh = plsc.VectorSubcoreMesh(core_axis_name="c", subcore_axis_name="s")

    @functools.partial(
        pl.kernel,
        mesh=mesh,
        out_type=(
            jax.ShapeDtypeStruct((S * B, U), jnp.float32),
            jax.ShapeDtypeStruct((T * B, U), jnp.float32),
        ),
        scratch_types=[
            pltpu.VMEM((sw,), jnp.int32),
            pltpu.VMEM((sw, U), jnp.float32),
            pltpu.VMEM((tw,), jnp.int32),
            pltpu.VMEM((tw, U), jnp.float32),
            pltpu.SemaphoreType.DMA,
            pltpu.SemaphoreType.DMA,
        ],
    )
    def gather(wtab_hbm, sidx_hbm, ctab_hbm, tidx_hbm, xsrc_hbm, xtgt_hbm,
               sidx_v, srows_v, tidx_v, trows_v, sem_s, sem_t):
        wid = lax.axis_index("s") * info.num_cores + lax.axis_index("c")
        sb = wid * sw
        tb = wid * tw
        pltpu.sync_copy(sidx_hbm.at[pl.ds(sb, sw)], sidx_v)
        pltpu.sync_copy(tidx_hbm.at[pl.ds(tb, tw)], tidx_v)
        cp_s = pltpu.async_copy(wtab_hbm.at[sidx_v], srows_v, sem_s)
        cp_t = pltpu.async_copy(ctab_hbm.at[tidx_v], trows_v, sem_t)
        cp_s.wait()
        pltpu.sync_copy(srows_v, xsrc_hbm.at[pl.ds(sb, sw)])
        cp_t.wait()
        pltpu.sync_copy(trows_v, xtgt_hbm.at[pl.ds(tb, tw)])

    return gather


def _xg_body(xsrc_ref, xtgt_ref, wxf_ref, wxb_ref, wxd_ref,
             xgf_ref, xgb_ref, xgd_ref):
    xs = xsrc_ref[:].astype(jnp.bfloat16)
    xgf_ref[:] = jnp.dot(
        xs, wxf_ref[:].astype(jnp.bfloat16), preferred_element_type=jnp.float32
    ).astype(jnp.bfloat16)
    xgb_ref[:] = jnp.dot(
        xs, wxb_ref[:].astype(jnp.bfloat16), preferred_element_type=jnp.float32
    ).astype(jnp.bfloat16)
    xgd_ref[:] = jnp.dot(
        xtgt_ref[:].astype(jnp.bfloat16), wxd_ref[:].astype(jnp.bfloat16),
        preferred_element_type=jnp.float32,
    ).astype(jnp.bfloat16)


def _xg_proj(xsrc, xtgt, wx_f, wx_b, wx_d):
    return pl.pallas_call(
        _xg_body,
        out_shape=(
            jax.ShapeDtypeStruct((S * B, 3 * U), jnp.bfloat16),
            jax.ShapeDtypeStruct((S * B, 3 * U), jnp.bfloat16),
            jax.ShapeDtypeStruct((T * B, 3 * M), jnp.bfloat16),
        ),
    )(xsrc, xtgt, wx_f, wx_b, wx_d)


_VBLK = 512


def _mega_body(
    xgf_ref, xgb_ref, xgd_ref, whf_ref, whb_ref, whd_ref,
    bf_ref, bb_ref, bd_ref, wc_ref, wo_ref, bo_ref,
    out_ref,
    hf_ref, hb_ref, h_ref, mem_ref, hs_ref, comb_ref,
):
    @pl.when(pl.program_id(0) == 0)
    def _():

        def enc_step(s, sp, xgf, xgb, hf, hb):
            hgf = jnp.dot(hf, whf_ref[:], preferred_element_type=jnp.float32)
            hgb = jnp.dot(hb, whb_ref[:], preferred_element_type=jnp.float32)
            bf = bf_ref[:]
            bb = bb_ref[:]
            zf = jax.nn.sigmoid(xgf[:, :U] + hgf[:, :U] + bf[:, :U])
            rf = jax.nn.sigmoid(
                xgf[:, U : 2 * U] + hgf[:, U : 2 * U] + bf[:, U : 2 * U]
            )
            nf = jnp.tanh(xgf[:, 2 * U :] + rf * (hgf[:, 2 * U :] + bf[:, 2 * U :]))
            hfn = (1.0 - zf) * nf + zf * hf
            zb = jax.nn.sigmoid(xgb[:, :U] + hgb[:, :U] + bb[:, :U])
            rb = jax.nn.sigmoid(
                xgb[:, U : 2 * U] + hgb[:, U : 2 * U] + bb[:, U : 2 * U]
            )
            nb = jnp.tanh(xgb[:, 2 * U :] + rb * (hgb[:, 2 * U :] + bb[:, 2 * U :]))
            hbn = (1.0 - zb) * nb + zb * hb
            mem_ref[:, pl.ds(s, 1), :U] = hfn[:, None, :]
            mem_ref[:, pl.ds(sp, 1), U:] = hbn[:, None, :]
            return hfn, hbn

        def enc_step4(i, c):
            hf, hb = c
            xgf4 = xgf_ref[pl.ds(i * 4 * B, 4 * B), :].astype(jnp.float32)
            xgb4 = xgb_ref[pl.ds((S // 4 - 1 - i) * 4 * B, 4 * B), :].astype(
                jnp.float32
            )
            for k in range(4):
                hf, hb = enc_step(
                    4 * i + k,
                    S - 1 - 4 * i - k,
                    xgf4[k * B : (k + 1) * B],
                    xgb4[(3 - k) * B : (4 - k) * B],
                    hf, hb,
                )
            return hf, hb

        z0 = jnp.zeros((B, U), jnp.float32)
        hfT, hbT = lax.fori_loop(0, S // 4, enc_step4, (z0, z0))
        h_ref[:, :U] = hfT
        h_ref[:, U:] = hbT

        def dec_step(t, xg, h):
            hg_zr = jnp.dot(
                h, whd_ref[:, : 2 * M], preferred_element_type=jnp.float32
            )
            hg_n = jnp.dot(
                h, whd_ref[:, 2 * M :], preferred_element_type=jnp.float32
            )
            bia = bd_ref[:]
            z = jax.nn.sigmoid(xg[:, :M] + hg_zr[:, :M] + bia[:, :M])
            r = jax.nn.sigmoid(xg[:, M : 2 * M] + hg_zr[:, M:] + bia[:, M : 2 * M])
            n = jnp.tanh(xg[:, 2 * M :] + r * (hg_n + bia[:, 2 * M :]))
            hn = (1.0 - z) * n + z * h
            hs_ref[:, pl.ds(t, 1), :] = hn[:, None, :]
            return hn

        def dec_step4(i, h):
            xg4 = xgd_ref[pl.ds(i * 4 * B, 4 * B), :].astype(jnp.float32)
            for k in range(4):
                h = dec_step(4 * i + k, xg4[k * B : (k + 1) * B], h)
            return h

        lax.fori_loop(0, T // 4, dec_step4, h_ref[:])

        wc_h = wc_ref[:M, :]
        wc_c = wc_ref[M:, :]
        for b in range(B):
            hsb = hs_ref[b]
            mb = mem_ref[b]
            scores = lax.dot_general(
                hsb, mb, (((1,), (1,)), ((), ())),
                preferred_element_type=jnp.float32,
            )
            mx = jnp.max(scores, axis=-1, keepdims=True)
            e = jnp.exp(scores - mx)
            attn = e / jnp.sum(e, axis=-1, keepdims=True)
            ctx = jnp.dot(attn, mb, preferred_element_type=jnp.float32)
            comb = jnp.tanh(
                jnp.dot(hsb, wc_h, preferred_element_type=jnp.float32)
                + jnp.dot(ctx, wc_c, preferred_element_type=jnp.float32)
            )
            comb_ref[b * T : (b + 1) * T, :] = comb.astype(jnp.bfloat16)

    out_ref[:] = (
        jnp.dot(
            comb_ref[:], wo_ref[:].astype(jnp.bfloat16),
            preferred_element_type=jnp.float32,
        )
        + bo_ref[:]
    )


def _mega(xgf, xgb, xgd, wh_f, wh_b, wh_d, b_f2, b_b2, b_d2, w_c, w_o, b_o2):
    nblk = pl.cdiv(V, _VBLK)
    full = lambda j: (0, 0)
    return pl.pallas_call(
        _mega_body,
        grid=(nblk,),
        in_specs=[
            pl.BlockSpec((S * B, 3 * U), full),
            pl.BlockSpec((S * B, 3 * U), full),
            pl.BlockSpec((T * B, 3 * M), full),
            pl.BlockSpec((U, 3 * U), full),
            pl.BlockSpec((U, 3 * U), full),
            pl.BlockSpec((M, 3 * M), full),
            pl.BlockSpec((1, 3 * U), full),
            pl.BlockSpec((1, 3 * U), full),
            pl.BlockSpec((1, 3 * M), full),
            pl.BlockSpec((2 * M, M), full),
            pl.BlockSpec((M, _VBLK), lambda j: (0, j)),
            pl.BlockSpec((1, _VBLK), lambda j: (0, j)),
        ],
        out_specs=pl.BlockSpec((B * T, _VBLK), lambda j: (0, j)),
        out_shape=jax.ShapeDtypeStruct((B * T, V), jnp.float32),
        scratch_shapes=[
            pltpu.VMEM((B, U), jnp.float32),
            pltpu.VMEM((B, U), jnp.float32),
            pltpu.VMEM((B, M), jnp.float32),
            pltpu.VMEM((B, S, M), jnp.float32),
            pltpu.VMEM((B, T, M), jnp.float32),
            pltpu.VMEM((B * T, M), jnp.bfloat16),
        ],
    )(xgf, xgb, xgd, wh_f, wh_b, wh_d, b_f2, b_b2, b_d2, w_c, w_o, b_o2)


def kernel(word_embed, code_embed, Wx_f, Wh_f, b_f, Wx_b, Wh_b, b_b,
           Wx_d, Wh_d, b_d, W_c, W_o, b_o, src_tokens, tgt_tokens):
    src_idx = src_tokens.T.reshape(-1)
    tgt_idx = tgt_tokens.T.reshape(-1)
    xsrc, xtgt = _make_sc_gather()(word_embed, src_idx, code_embed, tgt_idx)

    xgf, xgb, xgd = _xg_proj(xsrc, xtgt, Wx_f, Wx_b, Wx_d)
    logits = _mega(
        xgf, xgb, xgd, Wh_f, Wh_b, Wh_d,
        b_f.reshape(1, 3 * U), b_b.reshape(1, 3 * U), b_d.reshape(1, 3 * M),
        W_c.astype(jnp.bfloat16), W_o, b_o.reshape(1, V),
    )
    return logits.reshape(B, T, V)


# speedup vs baseline: 1.1273x; 1.0198x over previous
import functools

import jax
import jax.numpy as jnp
from jax import lax
from jax.experimental import pallas as pl
from jax.experimental.pallas import tpu as pltpu
from jax.experimental.pallas import tpu_sc as plsc

B = 8
S = 128
T = 64
U = 512
M = 2 * U
V = 8020


@functools.lru_cache(maxsize=None)
def _make_sc_gather():
    info = plsc.get_sparse_core_info()
    nw = info.num_cores * info.num_subcores
    sw = S * B // nw
    tw = T * B // nw
    mesh = plsc.VectorSubcoreMesh(core_axis_name="c", subcore_axis_name="s")

    @functools.partial(
        pl.kernel,
        mesh=mesh,
        out_type=(
            jax.ShapeDtypeStruct((S * B, U), jnp.float32),
            jax.ShapeDtypeStruct((T * B, U), jnp.float32),
        ),
        scratch_types=[
            pltpu.VMEM((sw,), jnp.int32),
            pltpu.VMEM((sw, U), jnp.float32),
            pltpu.VMEM((tw,), jnp.int32),
            pltpu.VMEM((tw, U), jnp.float32),
            pltpu.SemaphoreType.DMA,
            pltpu.SemaphoreType.DMA,
        ],
    )
    def gather(wtab_hbm, sidx_hbm, ctab_hbm, tidx_hbm, xsrc_hbm, xtgt_hbm,
               sidx_v, srows_v, tidx_v, trows_v, sem_s, sem_t):
        wid = lax.axis_index("s") * info.num_cores + lax.axis_index("c")
        sb = wid * sw
        tb = wid * tw
        pltpu.sync_copy(sidx_hbm.at[pl.ds(sb, sw)], sidx_v)
        pltpu.sync_copy(tidx_hbm.at[pl.ds(tb, tw)], tidx_v)
        cp_s = pltpu.async_copy(wtab_hbm.at[sidx_v], srows_v, sem_s)
        cp_t = pltpu.async_copy(ctab_hbm.at[tidx_v], trows_v, sem_t)
        cp_s.wait()
        pltpu.sync_copy(srows_v, xsrc_hbm.at[pl.ds(sb, sw)])
        cp_t.wait()
        pltpu.sync_copy(trows_v, xtgt_hbm.at[pl.ds(tb, tw)])

    return gather


def _xg_body(xsrc_ref, xtgt_ref, wxf_ref, wxb_ref, wxd_ref,
             xgf_ref, xgb_ref, xgd_ref):
    xs = xsrc_ref[:].astype(jnp.bfloat16)
    xgf_ref[:] = jnp.dot(
        xs, wxf_ref[:].astype(jnp.bfloat16), preferred_element_type=jnp.float32
    ).astype(jnp.bfloat16)
    xgb_ref[:] = jnp.dot(
        xs, wxb_ref[:].astype(jnp.bfloat16), preferred_element_type=jnp.float32
    ).astype(jnp.bfloat16)
    xgd_ref[:] = jnp.dot(
        xtgt_ref[:].astype(jnp.bfloat16), wxd_ref[:].astype(jnp.bfloat16),
        preferred_element_type=jnp.float32,
    ).astype(jnp.bfloat16)


def _xg_proj(xsrc, xtgt, wx_f, wx_b, wx_d):
    return pl.pallas_call(
        _xg_body,
        out_shape=(
            jax.ShapeDtypeStruct((S * B, 3 * U), jnp.bfloat16),
            jax.ShapeDtypeStruct((S * B, 3 * U), jnp.bfloat16),
            jax.ShapeDtypeStruct((T * B, 3 * M), jnp.bfloat16),
        ),
    )(xsrc, xtgt, wx_f, wx_b, wx_d)


_VBLK = 1024


def _mega_body(
    xgf_ref, xgb_ref, xgd_ref, whf_ref, whb_ref, whd_ref,
    bf_ref, bb_ref, bd_ref, wc_ref, wo_ref, bo_ref,
    out_ref,
    hf_ref, hb_ref, h_ref, mem_ref, hs_ref, comb_ref,
):
    @pl.when(pl.program_id(0) == 0)
    def _():

        def enc_step(s, sp, xgf, xgb, hf, hb):
            hgf = jnp.dot(hf, whf_ref[:], preferred_element_type=jnp.float32)
            hgb = jnp.dot(hb, whb_ref[:], preferred_element_type=jnp.float32)
            bf = bf_ref[:]
            bb = bb_ref[:]
            zf = jax.nn.sigmoid(xgf[:, :U] + hgf[:, :U] + bf[:, :U])
            rf = jax.nn.sigmoid(
                xgf[:, U : 2 * U] + hgf[:, U : 2 * U] + bf[:, U : 2 * U]
            )
            nf = jnp.tanh(xgf[:, 2 * U :] + rf * (hgf[:, 2 * U :] + bf[:, 2 * U :]))
            hfn = (1.0 - zf) * nf + zf * hf
            zb = jax.nn.sigmoid(xgb[:, :U] + hgb[:, :U] + bb[:, :U])
            rb = jax.nn.sigmoid(
                xgb[:, U : 2 * U] + hgb[:, U : 2 * U] + bb[:, U : 2 * U]
            )
            nb = jnp.tanh(xgb[:, 2 * U :] + rb * (hgb[:, 2 * U :] + bb[:, 2 * U :]))
            hbn = (1.0 - zb) * nb + zb * hb
            mem_ref[:, pl.ds(s, 1), :U] = hfn[:, None, :]
            mem_ref[:, pl.ds(sp, 1), U:] = hbn[:, None, :]
            return hfn, hbn

        def enc_step4(i, c):
            hf, hb = c
            xgf4 = xgf_ref[pl.ds(i * 4 * B, 4 * B), :].astype(jnp.float32)
            xgb4 = xgb_ref[pl.ds((S // 4 - 1 - i) * 4 * B, 4 * B), :].astype(
                jnp.float32
            )
            for k in range(4):
                hf, hb = enc_step(
                    4 * i + k,
                    S - 1 - 4 * i - k,
                    xgf4[k * B : (k + 1) * B],
                    xgb4[(3 - k) * B : (4 - k) * B],
                    hf, hb,
                )
            return hf, hb

        z0 = jnp.zeros((B, U), jnp.float32)
        hfT, hbT = lax.fori_loop(0, S // 4, enc_step4, (z0, z0))
        h_ref[:, :U] = hfT
        h_ref[:, U:] = hbT

        def dec_step(t, xg, h):
            hg_zr = jnp.dot(
                h, whd_ref[:, : 2 * M], preferred_element_type=jnp.float32
            )
            hg_n = jnp.dot(
                h, whd_ref[:, 2 * M :], preferred_element_type=jnp.float32
            )
            bia = bd_ref[:]
            z = jax.nn.sigmoid(xg[:, :M] + hg_zr[:, :M] + bia[:, :M])
            r = jax.nn.sigmoid(xg[:, M : 2 * M] + hg_zr[:, M:] + bia[:, M : 2 * M])
            n = jnp.tanh(xg[:, 2 * M :] + r * (hg_n + bia[:, 2 * M :]))
            hn = (1.0 - z) * n + z * h
            hs_ref[:, pl.ds(t, 1), :] = hn[:, None, :]
            return hn

        def dec_step4(i, h):
            xg4 = xgd_ref[pl.ds(i * 4 * B, 4 * B), :].astype(jnp.float32)
            for k in range(4):
                h = dec_step(4 * i + k, xg4[k * B : (k + 1) * B], h)
            return h

        lax.fori_loop(0, T // 4, dec_step4, h_ref[:])

        wc_h = wc_ref[:M, :]
        wc_c = wc_ref[M:, :]
        for b in range(B):
            hsb = hs_ref[b]
            mb = mem_ref[b]
            scores = lax.dot_general(
                hsb, mb, (((1,), (1,)), ((), ())),
                preferred_element_type=jnp.float32,
            )
            mx = jnp.max(scores, axis=-1, keepdims=True)
            e = jnp.exp(scores - mx)
            attn = e / jnp.sum(e, axis=-1, keepdims=True)
            ctx = jnp.dot(attn, mb, preferred_element_type=jnp.float32)
            comb = jnp.tanh(
                jnp.dot(hsb, wc_h, preferred_element_type=jnp.float32)
                + jnp.dot(ctx, wc_c, preferred_element_type=jnp.float32)
            )
            comb_ref[b * T : (b + 1) * T, :] = comb.astype(jnp.bfloat16)

    out_ref[:] = (
        jnp.dot(
            comb_ref[:], wo_ref[:].astype(jnp.bfloat16),
            preferred_element_type=jnp.float32,
        )
        + bo_ref[:]
    )


def _mega(xgf, xgb, xgd, wh_f, wh_b, wh_d, b_f2, b_b2, b_d2, w_c, w_o, b_o2):
    nblk = pl.cdiv(V, _VBLK)
    full = lambda j: (0, 0)
    return pl.pallas_call(
        _mega_body,
        grid=(nblk,),
        in_specs=[
            pl.BlockSpec((S * B, 3 * U), full),
            pl.BlockSpec((S * B, 3 * U), full),
            pl.BlockSpec((T * B, 3 * M), full),
            pl.BlockSpec((U, 3 * U), full),
            pl.BlockSpec((U, 3 * U), full),
            pl.BlockSpec((M, 3 * M), full),
            pl.BlockSpec((1, 3 * U), full),
            pl.BlockSpec((1, 3 * U), full),
            pl.BlockSpec((1, 3 * M), full),
            pl.BlockSpec((2 * M, M), full),
            pl.BlockSpec((M, _VBLK), lambda j: (0, j)),
            pl.BlockSpec((1, _VBLK), lambda j: (0, j)),
        ],
        out_specs=pl.BlockSpec((B * T, _VBLK), lambda j: (0, j)),
        out_shape=jax.ShapeDtypeStruct((B * T, V), jnp.float32),
        scratch_shapes=[
            pltpu.VMEM((B, U), jnp.float32),
            pltpu.VMEM((B, U), jnp.float32),
            pltpu.VMEM((B, M), jnp.float32),
            pltpu.VMEM((B, S, M), jnp.float32),
            pltpu.VMEM((B, T, M), jnp.float32),
            pltpu.VMEM((B * T, M), jnp.bfloat16),
        ],
    )(xgf, xgb, xgd, wh_f, wh_b, wh_d, b_f2, b_b2, b_d2, w_c, w_o, b_o2)


def kernel(word_embed, code_embed, Wx_f, Wh_f, b_f, Wx_b, Wh_b, b_b,
           Wx_d, Wh_d, b_d, W_c, W_o, b_o, src_tokens, tgt_tokens):
    src_idx = src_tokens.T.reshape(-1)
    tgt_idx = tgt_tokens.T.reshape(-1)
    xsrc, xtgt = _make_sc_gather()(word_embed, src_idx, code_embed, tgt_idx)

    xgf, xgb, xgd = _xg_proj(xsrc, xtgt, Wx_f, Wx_b, Wx_d)
    logits = _mega(
        xgf, xgb, xgd, Wh_f, Wh_b, Wh_d,
        b_f.reshape(1, 3 * U), b_b.reshape(1, 3 * U), b_d.reshape(1, 3 * M),
        W_c.astype(jnp.bfloat16), W_o, b_o.reshape(1, V),
    )
    return logits.reshape(B, T, V)
